# initial kernel scaffold (unmeasured)
import functools

import jax
import jax.numpy as jnp
from jax import lax
from jax.experimental import pallas as pl
from jax.experimental.pallas import tpu as pltpu

N_DEV = 8
N_EXP = 16
CAP = 128


def _a2a_moe_pallas(xsend, W1, W2):
    _, rows, d = xsend.shape

    def body(xsend_ref, w1_ref, w2_ref, out_ref,
             xrecv_ref, y_ref, send1, recv1, send2, recv2):
        my = lax.axis_index("i")

        bsem = pltpu.get_barrier_semaphore()
        for o in range(1, N_DEV):
            pl.semaphore_signal(
                bsem, inc=1,
                device_id=((my + o) % N_DEV,),
                device_id_type=pl.DeviceIdType.MESH,
            )
        pl.semaphore_wait(bsem, N_DEV - 1)

        pltpu.make_async_copy(
            xsend_ref.at[my], xrecv_ref.at[my], recv1.at[my]
        ).start()
        for o in range(1, N_DEV):
            dst = (my + o) % N_DEV
            pltpu.make_async_remote_copy(
                src_ref=xsend_ref.at[dst],
                dst_ref=xrecv_ref.at[my],
                send_sem=send1.at[dst],
                recv_sem=recv1.at[my],
                device_id=(dst,),
                device_id_type=pl.DeviceIdType.MESH,
            ).start()

        for s in range(N_DEV):
            pltpu.make_async_copy(
                xrecv_ref.at[s], xrecv_ref.at[s], recv1.at[s]
            ).wait()

        xv = xrecv_ref[...]
        x0 = xv[:, :CAP, :].reshape(N_DEV * CAP, d)
        x1 = xv[:, CAP:, :].reshape(N_DEV * CAP, d)
        h0 = jnp.maximum(
            jnp.dot(x0, w1_ref[0], preferred_element_type=jnp.float32), 0.0)
        y0 = jnp.dot(h0, w2_ref[0], preferred_element_type=jnp.float32)
        h1 = jnp.maximum(
            jnp.dot(x1, w1_ref[1], preferred_element_type=jnp.float32), 0.0)
        y1 = jnp.dot(h1, w2_ref[1], preferred_element_type=jnp.float32)
        y_ref[...] = jnp.concatenate(
            [y0.reshape(N_DEV, CAP, d), y1.reshape(N_DEV, CAP, d)], axis=1)

        pltpu.make_async_copy(
            y_ref.at[my], out_ref.at[my], recv2.at[my]
        ).start()
        for o in range(1, N_DEV):
            dst = (my + o) % N_DEV
            pltpu.make_async_remote_copy(
                src_ref=y_ref.at[dst],
                dst_ref=out_ref.at[my],
                send_sem=send2.at[dst],
                recv_sem=recv2.at[my],
                device_id=(dst,),
                device_id_type=pl.DeviceIdType.MESH,
            ).start()

        for o in range(1, N_DEV):
            dst = (my + o) % N_DEV
            pltpu.make_async_copy(
                xsend_ref.at[dst], xsend_ref.at[dst], send1.at[dst]
            ).wait()

        for s in range(N_DEV):
            pltpu.make_async_copy(
                out_ref.at[s], out_ref.at[s], recv2.at[s]
            ).wait()
        for o in range(1, N_DEV):
            dst = (my + o) % N_DEV
            pltpu.make_async_copy(
                y_ref.at[dst], y_ref.at[dst], send2.at[dst]
            ).wait()

    return pl.pallas_call(
        body,
        out_shape=jax.ShapeDtypeStruct((N_DEV, rows, d), jnp.float32),
        in_specs=[
            pl.BlockSpec(memory_space=pltpu.VMEM),
            pl.BlockSpec(memory_space=pltpu.VMEM),
            pl.BlockSpec(memory_space=pltpu.VMEM),
        ],
        out_specs=pl.BlockSpec(memory_space=pltpu.VMEM),
        scratch_shapes=[
            pltpu.VMEM((N_DEV, rows, d), jnp.float32),
            pltpu.VMEM((N_DEV, rows, d), jnp.float32),
            pltpu.SemaphoreType.DMA((N_DEV,)),
            pltpu.SemaphoreType.DMA((N_DEV,)),
            pltpu.SemaphoreType.DMA((N_DEV,)),
            pltpu.SemaphoreType.DMA((N_DEV,)),
        ],
        compiler_params=pltpu.CompilerParams(collective_id=0),
    )(xsend, W1, W2)


def kernel(x, assign, W1, W2):
    t, d = x.shape

    idx = jnp.stack([
        jnp.nonzero(assign == e, size=CAP, fill_value=t)[0]
        for e in range(N_EXP)
    ])

    xp = jnp.concatenate([x, jnp.zeros((1, d), x.dtype)], axis=0)
    xsend = xp[idx].reshape(N_DEV, 2 * CAP, d)

    out_slots = _a2a_moe_pallas(xsend, W1, W2)

    y16 = out_slots.reshape(N_EXP * CAP, d)
    return jnp.zeros((t, d), x.dtype).at[idx.reshape(-1)].set(y16, mode="drop")


# baseline (device time: 908675 ns/iter reference)
import functools

import jax
import jax.numpy as jnp
from jax import lax
from jax.experimental import pallas as pl
from jax.experimental.pallas import tpu as pltpu

N_DEV = 8
N_EXP = 16
CAP = 128


def _a2a_moe_pallas(xsend, W1, W2):
    _, rows, d = xsend.shape

    def body(xsend_ref, w1_ref, w2_ref, out_ref,
             xrecv_ref, y_ref, send1, recv1, send2, recv2):
        my = lax.axis_index("i")

        bsem = pltpu.get_barrier_semaphore()
        for o in range(1, N_DEV):
            pl.semaphore_signal(
                bsem, inc=1,
                device_id=((my + o) % N_DEV,),
                device_id_type=pl.DeviceIdType.MESH,
            )
        pl.semaphore_wait(bsem, N_DEV - 1)

        pltpu.make_async_copy(
            xsend_ref.at[my], xrecv_ref.at[my], recv1.at[my]
        ).start()
        for o in range(1, N_DEV):
            dst = (my + o) % N_DEV
            pltpu.make_async_remote_copy(
                src_ref=xsend_ref.at[dst],
                dst_ref=xrecv_ref.at[my],
                send_sem=send1.at[dst],
                recv_sem=recv1.at[my],
                device_id=(dst,),
                device_id_type=pl.DeviceIdType.MESH,
            ).start()

        for s in range(N_DEV):
            pltpu.make_async_copy(
                xrecv_ref.at[s], xrecv_ref.at[s], recv1.at[s]
            ).wait()

        for half in range(2):
            lo = half * CAP
            xh = xrecv_ref[:, lo:lo + CAP, :].reshape(N_DEV * CAP, d)
            h = jnp.maximum(
                jnp.dot(xh, w1_ref[half], preferred_element_type=jnp.float32),
                0.0).astype(jnp.bfloat16)
            yh = jnp.dot(h, w2_ref[half], preferred_element_type=jnp.float32)
            y_ref[:, lo:lo + CAP, :] = yh.astype(jnp.bfloat16).reshape(
                N_DEV, CAP, d)

        pltpu.make_async_copy(
            y_ref.at[my], out_ref.at[my], recv2.at[my]
        ).start()
        for o in range(1, N_DEV):
            dst = (my + o) % N_DEV
            pltpu.make_async_remote_copy(
                src_ref=y_ref.at[dst],
                dst_ref=out_ref.at[my],
                send_sem=send2.at[dst],
                recv_sem=recv2.at[my],
                device_id=(dst,),
                device_id_type=pl.DeviceIdType.MESH,
            ).start()

        for o in range(1, N_DEV):
            dst = (my + o) % N_DEV
            pltpu.make_async_copy(
                xsend_ref.at[dst], xsend_ref.at[dst], send1.at[dst]
            ).wait()

        for s in range(N_DEV):
            pltpu.make_async_copy(
                out_ref.at[s], out_ref.at[s], recv2.at[s]
            ).wait()
        for o in range(1, N_DEV):
            dst = (my + o) % N_DEV
            pltpu.make_async_copy(
                y_ref.at[dst], y_ref.at[dst], send2.at[dst]
            ).wait()

    return pl.pallas_call(
        body,
        out_shape=jax.ShapeDtypeStruct((N_DEV, rows, d), jnp.bfloat16),
        in_specs=[
            pl.BlockSpec(memory_space=pltpu.VMEM),
            pl.BlockSpec(memory_space=pltpu.VMEM),
            pl.BlockSpec(memory_space=pltpu.VMEM),
        ],
        out_specs=pl.BlockSpec(memory_space=pltpu.VMEM),
        scratch_shapes=[
            pltpu.VMEM((N_DEV, rows, d), jnp.bfloat16),
            pltpu.VMEM((N_DEV, rows, d), jnp.bfloat16),
            pltpu.SemaphoreType.DMA((N_DEV,)),
            pltpu.SemaphoreType.DMA((N_DEV,)),
            pltpu.SemaphoreType.DMA((N_DEV,)),
            pltpu.SemaphoreType.DMA((N_DEV,)),
        ],
        compiler_params=pltpu.CompilerParams(
            collective_id=0, vmem_limit_bytes=100 * 1024 * 1024),
    )(xsend, W1, W2)


def kernel(x, assign, W1, W2):
    t, d = x.shape

    idx = jnp.stack([
        jnp.nonzero(assign == e, size=CAP, fill_value=t)[0]
        for e in range(N_EXP)
    ])

    xp = jnp.concatenate([x, jnp.zeros((1, d), x.dtype)], axis=0)
    xsend = xp[idx].reshape(N_DEV, 2 * CAP, d).astype(jnp.bfloat16)

    out_slots = _a2a_moe_pallas(
        xsend, W1.astype(jnp.bfloat16), W2.astype(jnp.bfloat16))

    y16 = out_slots.reshape(N_EXP * CAP, d).astype(jnp.float32)
    return jnp.zeros((t, d), x.dtype).at[idx.reshape(-1)].set(y16, mode="drop")


# device time: 124990 ns/iter; 7.2700x vs baseline; 7.2700x over previous
import jax
import jax.numpy as jnp
from jax import lax
from jax.experimental import pallas as pl
from jax.experimental.pallas import tpu as pltpu

N_DEV = 8
N_EXP = 16
CAP = 128
ROWS = 2 * CAP


def _a2a_moe_pallas(xb, a_col, a_row, w1b, w2b):
    t, d = xb.shape

    def body(xb_ref, a_col_ref, a_row_ref, w1_ref, w2_ref, final_ref,
             xsend_ref, xrecv_ref, y_ref, outb_ref,
             send1, recv1, send2, recv2):
        my = lax.axis_index("i")

        bsem = pltpu.get_barrier_semaphore()
        for o in range(1, N_DEV):
            pl.semaphore_signal(
                bsem, inc=1,
                device_id=((my + o) % N_DEV,),
                device_id_type=pl.DeviceIdType.MESH,
            )

        ac = a_col_ref[...]
        ar = a_row_ref[...]
        row_i = lax.broadcasted_iota(jnp.int32, (t, t), 0)
        col_i = lax.broadcasted_iota(jnp.int32, (t, t), 1)
        earlier_same = jnp.where(
            (ac == ar) & (col_i < row_i), 1.0, 0.0)
        c = jnp.sum(earlier_same, axis=1, keepdims=True)
        rowmap = ((ac >> 1) * ROWS + (ac & 1) * CAP
                  + c.astype(jnp.int32))
        pt = jnp.where(
            rowmap == lax.broadcasted_iota(jnp.int32, (t, N_DEV * ROWS), 1),
            1.0, 0.0).astype(jnp.bfloat16)

        xsend_flat = lax.dot_general(
            pt, xb_ref[...],
            dimension_numbers=(((0,), (0,)), ((), ())),
            preferred_element_type=jnp.float32,
        ).astype(jnp.bfloat16)
        xsend_ref[...] = xsend_flat.reshape(N_DEV, ROWS, d)

        pl.semaphore_wait(bsem, N_DEV - 1)

        pltpu.make_async_copy(
            xsend_ref.at[my], xrecv_ref.at[my], recv1.at[my]
        ).start()
        for o in range(1, N_DEV):
            dst = (my + o) % N_DEV
            pltpu.make_async_remote_copy(
                src_ref=xsend_ref.at[dst],
                dst_ref=xrecv_ref.at[my],
                send_sem=send1.at[dst],
                recv_sem=recv1.at[my],
                device_id=(dst,),
                device_id_type=pl.DeviceIdType.MESH,
            ).start()
        for s in range(N_DEV):
            pltpu.make_async_copy(
                xrecv_ref.at[s], xrecv_ref.at[s], recv1.at[s]
            ).wait()

        for half in range(2):
            lo = half * CAP
            xh = xrecv_ref[:, lo:lo + CAP, :].reshape(N_DEV * CAP, d)
            h = jnp.maximum(
                jnp.dot(xh, w1_ref[half], preferred_element_type=jnp.float32),
                0.0).astype(jnp.bfloat16)
            yh = jnp.dot(h, w2_ref[half], preferred_element_type=jnp.float32)
            y_ref[:, lo:lo + CAP, :] = yh.astype(jnp.bfloat16).reshape(
                N_DEV, CAP, d)

        pltpu.make_async_copy(
            y_ref.at[my], outb_ref.at[my], recv2.at[my]
        ).start()
        for o in range(1, N_DEV):
            dst = (my + o) % N_DEV
            pltpu.make_async_remote_copy(
                src_ref=y_ref.at[dst],
                dst_ref=outb_ref.at[my],
                send_sem=send2.at[dst],
                recv_sem=recv2.at[my],
                device_id=(dst,),
                device_id_type=pl.DeviceIdType.MESH,
            ).start()

        for o in range(1, N_DEV):
            dst = (my + o) % N_DEV
            pltpu.make_async_copy(
                xsend_ref.at[dst], xsend_ref.at[dst], send1.at[dst]
            ).wait()

        for s in range(N_DEV):
            pltpu.make_async_copy(
                outb_ref.at[s], outb_ref.at[s], recv2.at[s]
            ).wait()

        final_ref[...] = jnp.dot(
            pt, outb_ref[...].reshape(N_DEV * ROWS, d),
            preferred_element_type=jnp.float32,
        ).astype(jnp.bfloat16)

        for o in range(1, N_DEV):
            dst = (my + o) % N_DEV
            pltpu.make_async_copy(
                y_ref.at[dst], y_ref.at[dst], send2.at[dst]
            ).wait()

    return pl.pallas_call(
        body,
        out_shape=jax.ShapeDtypeStruct((t, d), jnp.bfloat16),
        in_specs=[
            pl.BlockSpec(memory_space=pltpu.VMEM),
            pl.BlockSpec(memory_space=pltpu.VMEM),
            pl.BlockSpec(memory_space=pltpu.VMEM),
            pl.BlockSpec(memory_space=pltpu.VMEM),
            pl.BlockSpec(memory_space=pltpu.VMEM),
        ],
        out_specs=pl.BlockSpec(memory_space=pltpu.VMEM),
        scratch_shapes=[
            pltpu.VMEM((N_DEV, ROWS, d), jnp.bfloat16),
            pltpu.VMEM((N_DEV, ROWS, d), jnp.bfloat16),
            pltpu.VMEM((N_DEV, ROWS, d), jnp.bfloat16),
            pltpu.VMEM((N_DEV, ROWS, d), jnp.bfloat16),
            pltpu.SemaphoreType.DMA((N_DEV,)),
            pltpu.SemaphoreType.DMA((N_DEV,)),
            pltpu.SemaphoreType.DMA((N_DEV,)),
            pltpu.SemaphoreType.DMA((N_DEV,)),
        ],
        compiler_params=pltpu.CompilerParams(
            collective_id=0, vmem_limit_bytes=100 * 1024 * 1024),
    )(xb, a_col, a_row, w1b, w2b)


def kernel(x, assign, W1, W2):
    t = assign.shape[0]
    out_b = _a2a_moe_pallas(
        x.astype(jnp.bfloat16),
        assign.reshape(t, 1),
        assign.reshape(1, t),
        W1.astype(jnp.bfloat16),
        W2.astype(jnp.bfloat16),
    )
    return out_b.astype(jnp.float32)


# device time: 109126 ns/iter; 8.3268x vs baseline; 1.1454x over previous
import jax
import jax.numpy as jnp
from jax import lax
from jax.experimental import pallas as pl
from jax.experimental.pallas import tpu as pltpu

N_DEV = 8
N_EXP = 16
CAP = 128
ROWS = 2 * CAP


def _a2a_moe_pallas(xb, a_col, a_row, w1b, w2b):
    t, d = xb.shape

    def body(xb_ref, a_col_ref, a_row_ref, w1_ref, w2_ref, final_ref,
             xsend_ref, xrecv_ref, y_ref, outb_ref,
             send1, recv1, send2, recv2):
        my = lax.axis_index("i")

        bsem = pltpu.get_barrier_semaphore()
        for o in range(1, N_DEV):
            pl.semaphore_signal(
                bsem, inc=1,
                device_id=((my + o) % N_DEV,),
                device_id_type=pl.DeviceIdType.MESH,
            )

        ac = a_col_ref[...]
        ar = a_row_ref[...]
        row_i = lax.broadcasted_iota(jnp.int32, (t, t), 0)
        col_i = lax.broadcasted_iota(jnp.int32, (t, t), 1)
        earlier_same = jnp.where(
            (ac == ar) & (col_i < row_i), 1.0, 0.0)
        c = jnp.sum(earlier_same, axis=1, keepdims=True)
        rowmap = ((ac >> 1) * ROWS + (ac & 1) * CAP
                  + c.astype(jnp.int32))
        pt = jnp.where(
            rowmap == lax.broadcasted_iota(jnp.int32, (t, N_DEV * ROWS), 1),
            1.0, 0.0).astype(jnp.bfloat16)

        xsend_flat = lax.dot_general(
            pt, xb_ref[...],
            dimension_numbers=(((0,), (0,)), ((), ())),
            preferred_element_type=jnp.float32,
        ).astype(jnp.bfloat16)
        xsend_ref[...] = xsend_flat.reshape(N_DEV, ROWS, d)

        pl.semaphore_wait(bsem, N_DEV - 1)

        pltpu.make_async_copy(
            xsend_ref.at[my], xrecv_ref.at[my], recv1.at[my]
        ).start()
        for o in range(1, N_DEV):
            dst = (my + o) % N_DEV
            pltpu.make_async_remote_copy(
                src_ref=xsend_ref.at[dst],
                dst_ref=xrecv_ref.at[my],
                send_sem=send1.at[dst],
                recv_sem=recv1.at[my],
                device_id=(dst,),
                device_id_type=pl.DeviceIdType.MESH,
            ).start()
        for o in range(N_DEV):
            s = (my + o) % N_DEV
            pltpu.make_async_copy(
                xrecv_ref.at[s], xrecv_ref.at[s], recv1.at[s]
            ).wait()
            for half in range(2):
                lo = half * CAP
                xh = xrecv_ref[s, lo:lo + CAP, :]
                h = jnp.maximum(
                    jnp.dot(xh, w1_ref[half],
                            preferred_element_type=jnp.float32),
                    0.0).astype(jnp.bfloat16)
                yh = jnp.dot(h, w2_ref[half],
                             preferred_element_type=jnp.float32)
                y_ref[s, lo:lo + CAP, :] = yh.astype(jnp.bfloat16)
            if o == 0:
                pltpu.make_async_copy(
                    y_ref.at[my], outb_ref.at[my], recv2.at[my]
                ).start()
            else:
                pltpu.make_async_remote_copy(
                    src_ref=y_ref.at[s],
                    dst_ref=outb_ref.at[my],
                    send_sem=send2.at[s],
                    recv_sem=recv2.at[my],
                    device_id=(s,),
                    device_id_type=pl.DeviceIdType.MESH,
                ).start()

        for s in range(N_DEV):
            pltpu.make_async_copy(
                outb_ref.at[s], outb_ref.at[s], recv2.at[s]
            ).wait()

        final_ref[...] = jnp.dot(
            pt, outb_ref[...].reshape(N_DEV * ROWS, d),
            preferred_element_type=jnp.float32,
        ).astype(jnp.bfloat16)

        for o in range(1, N_DEV):
            dst = (my + o) % N_DEV
            pltpu.make_async_copy(
                xsend_ref.at[dst], xsend_ref.at[dst], send1.at[dst]
            ).wait()
            pltpu.make_async_copy(
                y_ref.at[dst], y_ref.at[dst], send2.at[dst]
            ).wait()

    return pl.pallas_call(
        body,
        out_shape=jax.ShapeDtypeStruct((t, d), jnp.bfloat16),
        in_specs=[
            pl.BlockSpec(memory_space=pltpu.VMEM),
            pl.BlockSpec(memory_space=pltpu.VMEM),
            pl.BlockSpec(memory_space=pltpu.VMEM),
            pl.BlockSpec(memory_space=pltpu.VMEM),
            pl.BlockSpec(memory_space=pltpu.VMEM),
        ],
        out_specs=pl.BlockSpec(memory_space=pltpu.VMEM),
        scratch_shapes=[
            pltpu.VMEM((N_DEV, ROWS, d), jnp.bfloat16),
            pltpu.VMEM((N_DEV, ROWS, d), jnp.bfloat16),
            pltpu.VMEM((N_DEV, ROWS, d), jnp.bfloat16),
            pltpu.VMEM((N_DEV, ROWS, d), jnp.bfloat16),
            pltpu.SemaphoreType.DMA((N_DEV,)),
            pltpu.SemaphoreType.DMA((N_DEV,)),
            pltpu.SemaphoreType.DMA((N_DEV,)),
            pltpu.SemaphoreType.DMA((N_DEV,)),
        ],
        compiler_params=pltpu.CompilerParams(
            collective_id=0, vmem_limit_bytes=100 * 1024 * 1024),
    )(xb, a_col, a_row, w1b, w2b)


def kernel(x, assign, W1, W2):
    t = assign.shape[0]
    out_b = _a2a_moe_pallas(
        x.astype(jnp.bfloat16),
        assign.reshape(t, 1),
        assign.reshape(1, t),
        W1.astype(jnp.bfloat16),
        W2.astype(jnp.bfloat16),
    )
    return out_b.astype(jnp.float32)
